# baseline (speedup 1.0000x reference)
import jax
import jax.numpy as jnp
from jax.experimental import pallas as pl
from jax.experimental.pallas import tpu as pltpu
from jax.experimental.pallas import tpu_sc as plsc

_RB = 4
_L = 16


def kernel(x, pe_table):
    B, S, H = x.shape

    mesh = plsc.VectorSubcoreMesh(core_axis_name="c", subcore_axis_name="s")

    @pl.kernel(out_type=jax.ShapeDtypeStruct((B, S, H), x.dtype), mesh=mesh)
    def pe_add_sc(x_hbm, pe_hbm, o_hbm):
        def body(x_vmem, pe_vmem, o_vmem):
            for r in range(_RB):

                @plsc.parallel_loop(0, H, step=_L, unroll=4)
                def _chunk(col, _r=r):
                    slc = pl.ds(col, _L)
                    pe_chunk = pe_vmem.at[_r].at[slc][...]
                    for b in range(B):
                        o_vmem.at[b].at[_r].at[slc][...] = (
                            x_vmem.at[b].at[_r].at[slc][...] + pe_chunk
                        )

        pltpu.emit_pipeline(
            body,
            grid=(S // _RB,),
            in_specs=[
                pl.BlockSpec((B, _RB, H), lambda i: (0, i, 0)),
                pl.BlockSpec((_RB, H), lambda i: (i, 0)),
            ],
            out_specs=[pl.BlockSpec((B, _RB, H), lambda i: (0, i, 0))],
            core_axis_name=("c", "s"),
            dimension_semantics=(pltpu.PARALLEL,),
        )(x_hbm, pe_hbm, o_hbm)

    return pe_add_sc(x, pe_table)
